# baseline (speedup 1.0000x reference)
import functools

import jax
import jax.numpy as jnp
from jax import lax
from jax.experimental import pallas as pl
from jax.experimental.pallas import tpu as pltpu
from jax.experimental.pallas import tpu_sc as plsc

_NUM_CORES = 2
_NUM_SUBCORES = 16
_NUM_WORKERS = _NUM_CORES * _NUM_SUBCORES
_LANES = 16

_BATCH = 4096
_MAXLEN = 200
_EMBED = 64
_SEQ_PER_W = _BATCH // _NUM_WORKERS
_HALF = _MAXLEN // 2


def _emb_body(x_hbm, tok_hbm, pos_hbm, out_hbm, idx_v, pos_v, rows_v, sem0, sem1):
    wid = lax.axis_index("s") * _NUM_CORES + lax.axis_index("c")
    base_seq = wid * _SEQ_PER_W

    pltpu.sync_copy(x_hbm.at[pl.ds(base_seq, _SEQ_PER_W)], idx_v)
    pltpu.sync_copy(pos_hbm, pos_v)

    sems = (sem0, sem1)

    def start_gather(s, b):
        for j in range(2):
            pltpu.async_copy(
                tok_hbm.at[idx_v.at[s, j]],
                rows_v.at[b, pl.ds(j * _HALF, _HALF)],
                sems[b],
            )

    def wait_gather(s, b):
        for j in range(2):
            pltpu.make_async_copy(
                tok_hbm.at[idx_v.at[s, j]],
                rows_v.at[b, pl.ds(j * _HALF, _HALF)],
                sems[b],
            ).wait()

    def pos_add(b):
        def row(m, carry):
            for l in range(_EMBED // _LANES):
                p = pos_v[m, pl.ds(l * _LANES, _LANES)]
                plsc.addupdate(rows_v.at[b, m, pl.ds(l * _LANES, _LANES)], p)
            return carry

        lax.fori_loop(0, _MAXLEN, row, 0, unroll=2)

    def writeback(s, b):
        pltpu.sync_copy(rows_v.at[b], out_hbm.at[base_seq + s])

    start_gather(0, 0)

    def outer(g, carry):
        s0 = 2 * g
        start_gather(s0 + 1, 1)
        wait_gather(s0, 0)
        pos_add(0)
        writeback(s0, 0)

        @pl.when(g < _SEQ_PER_W // 2 - 1)
        def _():
            start_gather(s0 + 2, 0)

        wait_gather(s0 + 1, 1)
        pos_add(1)
        writeback(s0 + 1, 1)
        return carry

    lax.fori_loop(0, _SEQ_PER_W // 2, outer, 0)


@jax.jit
def _emb(x3, token_table, pos_table):
    mesh = plsc.VectorSubcoreMesh(core_axis_name="c", subcore_axis_name="s")
    return pl.kernel(
        _emb_body,
        out_type=jax.ShapeDtypeStruct((_BATCH, _MAXLEN, _EMBED), jnp.float32),
        mesh=mesh,
        compiler_params=pltpu.CompilerParams(use_tc_tiling_on_sc=False),
        scratch_types=[
            pltpu.VMEM((_SEQ_PER_W, 2, _HALF), jnp.int32),
            pltpu.VMEM((_MAXLEN, _EMBED), jnp.float32),
            pltpu.VMEM((2, _MAXLEN, _EMBED), jnp.float32),
            pltpu.SemaphoreType.DMA,
            pltpu.SemaphoreType.DMA,
        ],
    )(x3, token_table, pos_table)


def kernel(x, token_table, pos_table):
    x3 = jnp.asarray(x, jnp.int32).reshape(_BATCH, 2, _HALF)
    return _emb(x3, token_table, pos_table)


# speedup vs baseline: 1.0236x; 1.0236x over previous
import functools

import jax
import jax.numpy as jnp
from jax import lax
from jax.experimental import pallas as pl
from jax.experimental.pallas import tpu as pltpu
from jax.experimental.pallas import tpu_sc as plsc

_NUM_CORES = 2
_NUM_SUBCORES = 16
_NUM_WORKERS = _NUM_CORES * _NUM_SUBCORES
_LANES = 16

_BATCH = 4096
_MAXLEN = 200
_EMBED = 64
_SEQ_PER_W = _BATCH // _NUM_WORKERS
_HALF = _MAXLEN // 2


_NBUF = 4
_GDEPTH = 2


def _emb_body(x_hbm, tok_hbm, pos_hbm, out_hbm, idx_v, pos_v, rows_v,
              g0, g1, g2, g3, w0, w1, w2, w3):
    gsems = (g0, g1, g2, g3)
    wsems = (w0, w1, w2, w3)
    wid = lax.axis_index("s") * _NUM_CORES + lax.axis_index("c")
    base_seq = wid * _SEQ_PER_W

    pltpu.sync_copy(x_hbm.at[pl.ds(base_seq, _SEQ_PER_W)], idx_v)
    pltpu.sync_copy(pos_hbm, pos_v)

    def start_gather(s, b):
        for j in range(2):
            pltpu.async_copy(
                tok_hbm.at[idx_v.at[s, j]],
                rows_v.at[b, pl.ds(j * _HALF, _HALF)],
                gsems[b],
            )

    def wait_gather(s, b):
        for j in range(2):
            pltpu.make_async_copy(
                tok_hbm.at[idx_v.at[s, j]],
                rows_v.at[b, pl.ds(j * _HALF, _HALF)],
                gsems[b],
            ).wait()

    def pos_add(b):
        def row(m, carry):
            for l in range(_EMBED // _LANES):
                p = pos_v[m, pl.ds(l * _LANES, _LANES)]
                plsc.addupdate(rows_v.at[b, m, pl.ds(l * _LANES, _LANES)], p)
            return carry

        lax.fori_loop(0, _MAXLEN, row, 0, unroll=2)

    def start_wb(s, b):
        pltpu.async_copy(rows_v.at[b], out_hbm.at[base_seq + s], wsems[b])

    def wait_wb(s, b):
        pltpu.make_async_copy(rows_v.at[b], out_hbm.at[base_seq + s],
                              wsems[b]).wait()

    for b in range(_GDEPTH):
        start_gather(b, b)

    def outer(g, carry):
        for b in range(_NBUF):
            s = g * _NBUF + b
            wait_gather(s, b)
            pos_add(b)
            start_wb(s, b)
            bn = (b + _GDEPTH) % _NBUF
            sn = s + _GDEPTH

            @pl.when(sn < _SEQ_PER_W)
            def _():
                @pl.when(sn >= _NBUF)
                def _():
                    wait_wb(sn - _NBUF, bn)
                start_gather(sn, bn)
        return carry

    lax.fori_loop(0, _SEQ_PER_W // _NBUF, outer, 0)

    for b in range(_NBUF):
        wait_wb(_SEQ_PER_W - _NBUF + b, b)


@jax.jit
def _emb(x3, token_table, pos_table):
    mesh = plsc.VectorSubcoreMesh(core_axis_name="c", subcore_axis_name="s")
    return pl.kernel(
        _emb_body,
        out_type=jax.ShapeDtypeStruct((_BATCH, _MAXLEN, _EMBED), jnp.float32),
        mesh=mesh,
        compiler_params=pltpu.CompilerParams(use_tc_tiling_on_sc=False),
        scratch_types=[
            pltpu.VMEM((_SEQ_PER_W, 2, _HALF), jnp.int32),
            pltpu.VMEM((_MAXLEN, _EMBED), jnp.float32),
            pltpu.VMEM((_NBUF, _MAXLEN, _EMBED), jnp.float32),
        ] + [pltpu.SemaphoreType.DMA] * (2 * _NBUF),
    )(x3, token_table, pos_table)


def kernel(x, token_table, pos_table):
    x3 = jnp.asarray(x, jnp.int32).reshape(_BATCH, 2, _HALF)
    return _emb(x3, token_table, pos_table)
